# baseline (speedup 1.0000x reference)
import functools

import jax
import jax.numpy as jnp
from jax import lax
from jax.experimental import pallas as pl
from jax.experimental.pallas import tpu as pltpu
from jax.experimental.pallas import tpu_sc as plsc

D = 64
SCALE = 8.0
SUB = 128
GPC = 2
CHUNK = SUB * GPC
NBUF = 4
BW = 256

_info = plsc.get_sparse_core_info()
_NC, _NS = _info.num_cores, _info.num_subcores
NW = _NC * _NS

_MESH = dict(core_axis_name="c", subcore_axis_name="s")


def _wid():
    return lax.axis_index("s") * _NC + lax.axis_index("c")


def _make_detranspose(n_rows: int):
    full_blocks = n_rows // 128
    tail = n_rows - full_blocks * 128
    per_w = (full_blocks + NW - 1) // NW

    @functools.partial(
        pl.kernel,
        out_type=jax.ShapeDtypeStruct(((n_rows + 1) // 2, 128), jnp.float32),
        mesh=plsc.VectorSubcoreMesh(**_MESH),
        scratch_types=[
            pltpu.VMEM((D, 128), jnp.float32),
            pltpu.VMEM((D, 128), jnp.float32),
            pltpu.SemaphoreType.DMA,
        ],
        compiler_params=pltpu.CompilerParams(use_tc_tiling_on_sc=True, needs_layout_passes=False),
    )
    def detranspose(tab_t, out, src_v, dst_v, sem):
        w = _wid()
        lo = w * per_w
        hi = lax.min(lo + per_w, full_blocks)
        iota = lax.iota(jnp.int32, 16)
        zero = jnp.zeros((16,), jnp.int32)

        def do_block(r0, n_r):
            r0 = pl.multiple_of(r0, 128)
            for ct in range(D // 8):
                pltpu.async_copy(
                    tab_t.at[pl.ds(8 * ct, 8), pl.ds(r0, n_r)],
                    src_v.at[pl.ds(8 * ct, 8), pl.ds(0, n_r)],
                    sem,
                ).wait()

            def tr(p, carry):
                for u in range(8):
                    v = plsc.load_gather(
                        src_v,
                        [iota + 16 * (u % 4), zero + (2 * p + (1 if u >= 4 else 0))],
                    )
                    dst_v[p, pl.ds(16 * u, 16)] = v
                return carry

            lax.fori_loop(0, n_r // 2, tr, None)
            pltpu.sync_copy(dst_v.at[pl.ds(0, n_r // 2), :],
                            out.at[pl.ds(pl.multiple_of(r0 // 2, 64),
                                         n_r // 2), :])

        def body(j, carry):
            do_block(j * 128, 128)
            return carry

        lax.fori_loop(lo, hi, body, None)
        if tail:
            @pl.when(w == NW - 1)
            def _():
                do_block(full_blocks * 128, tail)

    return detranspose


def _make_gather(n_lookups: int):
    per_w = n_lookups // NW
    subs_per_w = per_w // SUB
    n_chunks = per_w // CHUNK

    @functools.partial(
        pl.kernel,
        out_type=jax.ShapeDtypeStruct((n_lookups, D), jnp.float32),
        mesh=plsc.VectorSubcoreMesh(**_MESH),
        scratch_types=[
            pltpu.VMEM((subs_per_w, SUB), jnp.int32),
            pltpu.VMEM((NBUF, CHUNK, D), jnp.float32),
            pltpu.SemaphoreType.DMA((NBUF,)),
            pltpu.SemaphoreType.DMA((NBUF,)),
        ],
        compiler_params=pltpu.CompilerParams(use_tc_tiling_on_sc=False),
    )
    def gather_k(idx_hbm, table_hbm, out_hbm, idx_v, bufs, gsem, ssem):
        w = _wid()
        base = w * per_w
        pltpu.sync_copy(idx_hbm.at[pl.ds(w * subs_per_w, subs_per_w)], idx_v)

        def gather(g, b):
            return [
                pltpu.make_async_copy(
                    table_hbm.at[idx_v.at[g * GPC + j]],
                    bufs.at[b].at[pl.ds(j * SUB, SUB)],
                    gsem.at[b],
                )
                for j in range(GPC)
            ]

        def scatter(g, b):
            return pltpu.make_async_copy(
                bufs.at[b], out_hbm.at[pl.ds(base + g * CHUNK, CHUNK)],
                ssem.at[b],
            )

        for b in range(NBUF - 1):
            for cp in gather(b, b):
                cp.start()

        def outer(k, carry):
            g0 = k * NBUF
            for b in range(NBUF):
                g = g0 + b
                for cp in gather(g, b):
                    cp.wait()
                scatter(g, b).start()
                pb = (b - 1) % NBUF

                @pl.when(g > 0)
                def _():
                    scatter(g - 1, pb).wait()

                @pl.when(g + NBUF - 1 < n_chunks)
                def _():
                    for cp in gather(g + NBUF - 1, pb):
                        cp.start()

            return carry

        lax.fori_loop(0, n_chunks // NBUF, outer, None)
        scatter(n_chunks - 1, (n_chunks - 1) % NBUF).wait()

    return gather_k


def _make_retile(n_lookups: int, n_l: int, n_b: int):
    per_w = n_lookups // NW
    blocks_per_w = per_w // BW

    @functools.partial(
        pl.kernel,
        out_type=jax.ShapeDtypeStruct((n_l, D, n_b), jnp.float32),
        mesh=plsc.VectorSubcoreMesh(**_MESH),
        scratch_types=[
            pltpu.VMEM((BW // 2, 128), jnp.float32),
            pltpu.VMEM((D, BW), jnp.float32),
            pltpu.SemaphoreType.DMA,
        ],
        compiler_params=pltpu.CompilerParams(use_tc_tiling_on_sc=True, needs_layout_passes=False),
    )
    def retile(rows_hbm, out, iv, ov, sem):
        w = _wid()
        n_base = w * per_w
        iota = lax.iota(jnp.int32, 16)
        halfiota = lax.shift_right_logical(iota, 1)
        colpat = (iota & 1) * D

        def blk(k, carry):
            n0 = n_base + k * BW
            l = lax.shift_right_logical(n0, 12)
            b0 = pl.multiple_of(n0 & (n_b - 1), BW)
            pltpu.sync_copy(
                rows_hbm.at[pl.ds(pl.multiple_of(
                    lax.shift_right_logical(n0, 1), BW // 2), BW // 2), :], iv)

            def tc(c, c2):
                colv = colpat + c
                for t in range(BW // 16):
                    v = plsc.load_gather(iv, [halfiota + 8 * t, colv])
                    ov[c, pl.ds(16 * t, 16)] = v * SCALE
                return c2

            lax.fori_loop(0, D, tc, None)
            for ct in range(D // 8):
                pltpu.async_copy(
                    ov.at[pl.ds(8 * ct, 8), :],
                    out.at[l, pl.ds(8 * ct, 8), pl.ds(b0, BW)],
                    sem,
                ).wait()
            return carry

        lax.fori_loop(0, blocks_per_w, blk, None)

    return retile


def kernel(input_, table):
    l, b = input_.shape
    n = l * b
    v, d = table.shape
    tab_t = table.T
    tab_pairs = _make_detranspose(v)(tab_t)
    tab_lin = tab_pairs.reshape(v, d)
    idx2d = input_.reshape(n // SUB, SUB)
    rows = _make_gather(n)(idx2d, tab_lin)
    out_t = _make_retile(n, l, b)(rows.reshape(n // 2, 128))
    return out_t.transpose(0, 2, 1)


# speedup vs baseline: 1.4470x; 1.4470x over previous
import functools

import jax
import jax.numpy as jnp
from jax import lax
from jax.experimental import pallas as pl
from jax.experimental.pallas import tpu as pltpu
from jax.experimental.pallas import tpu_sc as plsc

D = 64
SCALE = 8.0
SUB = 128
GPC = 2
CHUNK = SUB * GPC
NBUF = 4
BW = 256

_info = plsc.get_sparse_core_info()
_NC, _NS = _info.num_cores, _info.num_subcores
NW = _NC * _NS

_MESH = dict(core_axis_name="c", subcore_axis_name="s")


def _wid():
    return lax.axis_index("s") * _NC + lax.axis_index("c")


def _make_detranspose(n_rows: int):
    full_blocks = n_rows // 128
    tail = n_rows - full_blocks * 128
    per_w = (full_blocks + NW - 1) // NW

    @functools.partial(
        pl.kernel,
        out_type=jax.ShapeDtypeStruct(((n_rows + 1) // 2, 128), jnp.float32),
        mesh=plsc.VectorSubcoreMesh(**_MESH),
        scratch_types=[
            pltpu.VMEM((D, 128), jnp.float32),
            pltpu.VMEM((D, 128), jnp.float32),
            pltpu.SemaphoreType.DMA,
        ],
        compiler_params=pltpu.CompilerParams(use_tc_tiling_on_sc=True, needs_layout_passes=False),
    )
    def detranspose(tab_t, out, src_v, dst_v, sem):
        w = _wid()
        lo = w * per_w
        hi = lax.min(lo + per_w, full_blocks)
        iota = lax.iota(jnp.int32, 16)
        zero = jnp.zeros((16,), jnp.int32)
        iotas = [iota + 16 * q for q in range(4)]

        def do_block(r0, n_r):
            r0 = pl.multiple_of(r0, 128)
            for ct in range(D // 8):
                pltpu.async_copy(
                    tab_t.at[pl.ds(8 * ct, 8), pl.ds(r0, n_r)],
                    src_v.at[pl.ds(8 * ct, 8), pl.ds(0, n_r)],
                    sem,
                ).wait()

            @plsc.parallel_loop(0, n_r // 2, unroll=8)
            def tr(p):
                cols_even = zero + 2 * p
                cols_odd = cols_even + 1
                for u in range(8):
                    v = plsc.load_gather(
                        src_v,
                        [iotas[u % 4], cols_odd if u >= 4 else cols_even],
                    )
                    dst_v[p, pl.ds(16 * u, 16)] = v
            pltpu.sync_copy(dst_v.at[pl.ds(0, n_r // 2), :],
                            out.at[pl.ds(pl.multiple_of(r0 // 2, 64),
                                         n_r // 2), :])

        def body(j, carry):
            do_block(j * 128, 128)
            return carry

        lax.fori_loop(lo, hi, body, None)
        if tail:
            @pl.when(w == NW - 1)
            def _():
                do_block(full_blocks * 128, tail)

    return detranspose


def _make_gather(n_lookups: int):
    per_w = n_lookups // NW
    subs_per_w = per_w // SUB
    n_chunks = per_w // CHUNK

    @functools.partial(
        pl.kernel,
        out_type=jax.ShapeDtypeStruct((n_lookups, D), jnp.float32),
        mesh=plsc.VectorSubcoreMesh(**_MESH),
        scratch_types=[
            pltpu.VMEM((subs_per_w, SUB), jnp.int32),
            pltpu.VMEM((NBUF, CHUNK, D), jnp.float32),
            pltpu.SemaphoreType.DMA((NBUF,)),
            pltpu.SemaphoreType.DMA((NBUF,)),
        ],
        compiler_params=pltpu.CompilerParams(use_tc_tiling_on_sc=False),
    )
    def gather_k(idx_hbm, table_hbm, out_hbm, idx_v, bufs, gsem, ssem):
        w = _wid()
        base = w * per_w
        pltpu.sync_copy(idx_hbm.at[pl.ds(w * subs_per_w, subs_per_w)], idx_v)

        def gather(g, b):
            return [
                pltpu.make_async_copy(
                    table_hbm.at[idx_v.at[g * GPC + j]],
                    bufs.at[b].at[pl.ds(j * SUB, SUB)],
                    gsem.at[b],
                )
                for j in range(GPC)
            ]

        def scatter(g, b):
            return pltpu.make_async_copy(
                bufs.at[b], out_hbm.at[pl.ds(base + g * CHUNK, CHUNK)],
                ssem.at[b],
            )

        for b in range(NBUF - 1):
            for cp in gather(b, b):
                cp.start()

        def outer(k, carry):
            g0 = k * NBUF
            for b in range(NBUF):
                g = g0 + b
                for cp in gather(g, b):
                    cp.wait()
                scatter(g, b).start()
                pb = (b - 1) % NBUF

                @pl.when(g > 0)
                def _():
                    scatter(g - 1, pb).wait()

                @pl.when(g + NBUF - 1 < n_chunks)
                def _():
                    for cp in gather(g + NBUF - 1, pb):
                        cp.start()

            return carry

        lax.fori_loop(0, n_chunks // NBUF, outer, None)
        scatter(n_chunks - 1, (n_chunks - 1) % NBUF).wait()

    return gather_k


def _make_retile(n_lookups: int, n_l: int, n_b: int):
    per_w = n_lookups // NW
    blocks_per_w = per_w // BW

    @functools.partial(
        pl.kernel,
        out_type=jax.ShapeDtypeStruct((n_l, D, n_b), jnp.float32),
        mesh=plsc.VectorSubcoreMesh(**_MESH),
        scratch_types=[
            pltpu.VMEM((BW // 2, 128), jnp.float32),
            pltpu.VMEM((D, BW), jnp.float32),
            pltpu.SemaphoreType.DMA,
        ],
        compiler_params=pltpu.CompilerParams(use_tc_tiling_on_sc=True, needs_layout_passes=False),
    )
    def retile(rows_hbm, out, iv, ov, sem):
        w = _wid()
        n_base = w * per_w
        iota = lax.iota(jnp.int32, 16)
        halfiota = lax.shift_right_logical(iota, 1)
        colpat = (iota & 1) * D
        rowpats = [halfiota + 8 * t for t in range(BW // 16)]

        def blk(k, carry):
            n0 = n_base + k * BW
            l = lax.shift_right_logical(n0, 12)
            b0 = pl.multiple_of(n0 & (n_b - 1), BW)
            pltpu.sync_copy(
                rows_hbm.at[pl.ds(pl.multiple_of(
                    lax.shift_right_logical(n0, 1), BW // 2), BW // 2), :], iv)

            @plsc.parallel_loop(0, D, unroll=4)
            def tc(c):
                colv = colpat + c
                for t in range(BW // 16):
                    v = plsc.load_gather(iv, [rowpats[t], colv])
                    ov[c, pl.ds(16 * t, 16)] = v * SCALE
            for ct in range(D // 8):
                pltpu.async_copy(
                    ov.at[pl.ds(8 * ct, 8), :],
                    out.at[l, pl.ds(8 * ct, 8), pl.ds(b0, BW)],
                    sem,
                ).wait()
            return carry

        lax.fori_loop(0, blocks_per_w, blk, None)

    return retile


def kernel(input_, table):
    l, b = input_.shape
    n = l * b
    v, d = table.shape
    tab_t = table.T
    tab_pairs = _make_detranspose(v)(tab_t)
    tab_lin = tab_pairs.reshape(v, d)
    idx2d = input_.reshape(n // SUB, SUB)
    rows = _make_gather(n)(idx2d, tab_lin)
    out_t = _make_retile(n, l, b)(rows.reshape(n // 2, 128))
    return out_t.transpose(0, 2, 1)
